# Initial kernel scaffold; baseline (speedup 1.0000x reference)
import jax
import jax.numpy as jnp
from jax.experimental import pallas as pl


def kernel(items, A, inputs, masks, alias_inputs, item_emb_w, intent_emb_w, pos_emb_w, lq_w, lk_w, la_w, lb_w, la1_w, la2_w, lb1_w, lb2_w, l1_w, l1_b, l2_w, l2_b, l3_w):
    raise NotImplementedError("write your pallas kernel here")


# speedup vs baseline: 57.8669x; 57.8669x over previous
import functools

import jax
import jax.numpy as jnp
import numpy as np
from jax.experimental import pallas as pl
from jax.experimental.pallas import tpu as pltpu

N_ITEMS = 100000
N_INTENTS = 512
H = 64
B = 64
L = 50
NU = 50
ALPHA = 0.5

RB1 = 1000
NBLK1 = N_ITEMS // RB1
RB3 = 2048
NBLK3 = (N_ITEMS + RB3 - 1) // RB3

_F32 = jnp.float32
_NEG = -3e38


def _dot(a, b, ca, cb):
    return jax.lax.dot_general(
        a, b, (((ca,), (cb,)), ((), ())), preferred_element_type=_F32)


def _leaky(x):
    return jnp.where(x >= 0, x, 0.2 * x)


def _graph_kernel(item_ref, iw_ref, k_ref, lq_ref, la_ref,
                  topi_ref, inew_ref, m_ref, s_ref, v_ref):
    i = pl.program_id(0)

    @pl.when(i == 0)
    def _init():
        m_ref[...] = jnp.full((1, N_INTENTS), _NEG, _F32)
        s_ref[...] = jnp.zeros((1, N_INTENTS), _F32)
        v_ref[...] = jnp.zeros((H, N_INTENTS), _F32)

    blk = item_ref[...]
    w = blk / jnp.sqrt(jnp.sum(blk * blk, axis=1, keepdims=True))
    iw = iw_ref[...]
    kmat = k_ref[...]

    q = _dot(w, lq_ref[...], 1, 1)
    logits = _dot(q, kmat, 1, 1)

    colid = jax.lax.broadcasted_iota(jnp.int32, (RB1, N_INTENTS), 1)
    l = logits
    sel_any = jnp.zeros((RB1, N_INTENTS), jnp.bool_)
    idxs = []
    for _ in range(3):
        mk = jnp.max(l, axis=1, keepdims=True)
        cand = jnp.where(l == mk, colid, N_INTENTS)
        idx = jnp.min(cand, axis=1, keepdims=True)
        sel = colid == idx
        sel_any = jnp.logical_or(sel_any, sel)
        l = jnp.where(sel, _NEG, l)
        idxs.append(idx)
    topi_ref[...] = jnp.concatenate(idxs, axis=1)

    p = _dot(w * la_ref[...], iw, 1, 1)
    e = _leaky(p)
    em = jnp.where(sel_any, e, _NEG)
    bm = jnp.max(em, axis=0, keepdims=True)
    m_old = m_ref[...]
    m_new = jnp.maximum(m_old, bm)
    eexp = jnp.where(sel_any, jnp.exp(e - m_new), 0.0)
    scale = jnp.exp(m_old - m_new)
    s_ref[...] = s_ref[...] * scale + jnp.sum(eexp, axis=0, keepdims=True)
    v_ref[...] = v_ref[...] * scale + _dot(w, eexp, 0, 0)
    m_ref[...] = m_new

    @pl.when(i == NBLK1 - 1)
    def _fin():
        s = s_ref[...]
        inew_ref[...] = jnp.where(s > 0, v_ref[...] / s, 0.0)


def _run_graph(item_emb_w, iw, kmat, lq_w, la_w):
    topi, inew = pl.pallas_call(
        _graph_kernel,
        grid=(NBLK1,),
        in_specs=[
            pl.BlockSpec((RB1, H), lambda i: (i, 0)),
            pl.BlockSpec((N_INTENTS, H), lambda i: (0, 0)),
            pl.BlockSpec((N_INTENTS, H), lambda i: (0, 0)),
            pl.BlockSpec((H, H), lambda i: (0, 0)),
            pl.BlockSpec((1, H), lambda i: (0, 0)),
        ],
        out_specs=[
            pl.BlockSpec((RB1, 3), lambda i: (i, 0)),
            pl.BlockSpec((H, N_INTENTS), lambda i: (0, 0)),
        ],
        out_shape=[
            jax.ShapeDtypeStruct((N_ITEMS, 3), jnp.int32),
            jax.ShapeDtypeStruct((H, N_INTENTS), _F32),
        ],
        scratch_shapes=[
            pltpu.VMEM((1, N_INTENTS), _F32),
            pltpu.VMEM((1, N_INTENTS), _F32),
            pltpu.VMEM((H, N_INTENTS), _F32),
        ],
    )(item_emb_w, iw, kmat, lq_w, la_w)
    return topi, inew


def _session_kernel(hi_ref, win_ref, is0_ref, is1_ref, is2_ref, a_ref, oh_ref,
                    mcol_ref, lb_ref, la1_ref, la2_ref, lb1_ref, lb2_ref,
                    l1w_ref, l1b_ref, l2w_ref, l2b_ref, l3w_ref, pos_ref,
                    hf_ref):
    hi_raw = hi_ref[0]
    hi = hi_raw / jnp.sqrt(jnp.sum(hi_raw * hi_raw, axis=1, keepdims=True))
    win_raw = win_ref[0]
    win = win_raw / jnp.sqrt(jnp.sum(win_raw * win_raw, axis=1, keepdims=True))
    mcol = mcol_ref[0]
    msum = jnp.sum(mcol)
    hs = jnp.sum(win * mcol, axis=0, keepdims=True) / msum

    lb = lb_ref[...]
    is0, is1, is2 = is0_ref[0], is1_ref[0], is2_ref[0]
    e0 = _leaky(jnp.sum(hi * is0 * lb, axis=1, keepdims=True))
    e1 = _leaky(jnp.sum(hi * is1 * lb, axis=1, keepdims=True))
    e2 = _leaky(jnp.sum(hi * is2 * lb, axis=1, keepdims=True))
    em = jnp.maximum(e0, jnp.maximum(e1, e2))
    x0, x1, x2 = jnp.exp(e0 - em), jnp.exp(e1 - em), jnp.exp(e2 - em)
    xs = x0 + x1 + x2
    item_nei = (x0 * is0 + x1 * is1 + x2 * is2) / xs
    h_intent = ALPHA * hi + (1.0 - ALPHA) * item_nei

    a = a_ref[0]
    c1 = _dot(hs * lb1_ref[...], hi, 1, 1)
    c2 = _dot(hs * lb2_ref[...], hi, 1, 1)
    att1 = _leaky(_dot(hi * la1_ref[...], hi, 1, 1) + c1)
    att2 = _leaky(_dot(hi * la2_ref[...], hi, 1, 1) + c2)
    al = jnp.where(a == 1.0, att1, jnp.where(a == 2.0, att2, -9e15))
    almax = jnp.max(al, axis=1, keepdims=True)
    ale = jnp.exp(al - almax)
    al = ale / jnp.sum(ale, axis=1, keepdims=True)
    h_local = _dot(al, hi, 1, 0)

    output = h_local + h_intent + h_local * h_intent
    sessions = _dot(oh_ref[0], output, 1, 0)

    spe = pos_ref[0:L, :]
    hi2 = sessions + spe
    hs2 = jnp.sum(hi2 * mcol, axis=0, keepdims=True) / msum
    q1 = _dot(hi2, l1w_ref[...], 1, 1) + l1b_ref[...]
    q2 = _dot(hs2, l2w_ref[...], 1, 1) + l2b_ref[...]
    alpha_r = _dot(jax.nn.sigmoid(q1 + q2), l3w_ref[...], 1, 1)
    hf = jnp.sum(alpha_r * sessions * mcol, axis=0, keepdims=True)
    pos_n = pos_ref[L:L + 1, :] * (
        jnp.sum(spe * mcol, axis=0, keepdims=True) / msum)
    hf_ref[0] = hf + pos_n


def _run_session(hi_raw, win_raw, is0, is1, is2, a, oh, mcol,
                 lb_w, la1_w, la2_w, lb1_w, lb2_w,
                 l1_w, l1_b, l2_w, l2_b, l3_w, pos_emb_w):
    full = lambda s: pl.BlockSpec(s, lambda b: tuple(0 for _ in s))
    per_b3 = lambda s1, s2: pl.BlockSpec((1, s1, s2), lambda b: (b, 0, 0))
    hf = pl.pallas_call(
        _session_kernel,
        grid=(B,),
        in_specs=[
            per_b3(NU, H), per_b3(L, H),
            per_b3(NU, H), per_b3(NU, H), per_b3(NU, H),
            per_b3(NU, NU), per_b3(L, NU), per_b3(L, 1),
            full((1, H)), full((1, H)), full((1, H)), full((1, H)),
            full((1, H)),
            full((H, H)), full((1, H)), full((H, H)), full((1, H)),
            full((1, H)), full((L + 1, H)),
        ],
        out_specs=pl.BlockSpec((1, 1, H), lambda b: (b, 0, 0)),
        out_shape=jax.ShapeDtypeStruct((B, 1, H), _F32),
    )(hi_raw, win_raw, is0, is1, is2, a, oh, mcol,
      lb_w, la1_w, la2_w, lb1_w, lb2_w,
      l1_w, l1_b, l2_w, l2_b, l3_w, pos_emb_w)
    return hf.reshape(B, H)


def _scores_kernel(item_ref, hf_ref, out_ref):
    blk = item_ref[...]
    w = blk / jnp.sqrt(jnp.sum(blk * blk, axis=1, keepdims=True))
    out_ref[...] = _dot(hf_ref[...], w, 1, 1)


def _run_scores(item_emb_w, hf):
    full = pl.pallas_call(
        _scores_kernel,
        grid=(NBLK3,),
        in_specs=[
            pl.BlockSpec((RB3, H), lambda j: (j, 0)),
            pl.BlockSpec((B, H), lambda j: (0, 0)),
        ],
        out_specs=pl.BlockSpec((B, RB3), lambda j: (0, j)),
        out_shape=jax.ShapeDtypeStruct((B, N_ITEMS), _F32),
    )(item_emb_w, hf)
    return full[:, 1:]


def kernel(items, A, inputs, masks, alias_inputs, item_emb_w, intent_emb_w,
           pos_emb_w, lq_w, lk_w, la_w, lb_w, la1_w, la2_w, lb1_w, lb2_w,
           l1_w, l1_b, l2_w, l2_b, l3_w):
    iw = intent_emb_w / jnp.linalg.norm(intent_emb_w, axis=1, keepdims=True)
    kmat = (iw @ lk_w.T) * np.float32(1.0 / np.sqrt(H))

    topi, inew = _run_graph(item_emb_w, iw, kmat, lq_w, la_w)
    intent_new = inew.T

    items_f = items.reshape(-1)
    hi_raw = jnp.take(item_emb_w, items_f, axis=0).reshape(B, NU, H)
    win_raw = jnp.take(item_emb_w, inputs.reshape(-1), axis=0).reshape(B, L, H)
    ti = jnp.take(topi, items_f, axis=0)
    is0 = jnp.take(intent_new, ti[:, 0], axis=0).reshape(B, NU, H)
    is1 = jnp.take(intent_new, ti[:, 1], axis=0).reshape(B, NU, H)
    is2 = jnp.take(intent_new, ti[:, 2], axis=0).reshape(B, NU, H)
    oh = jax.nn.one_hot(alias_inputs, NU, dtype=_F32)
    mcol = masks.reshape(B, L, 1)

    hf = _run_session(hi_raw, win_raw, is0, is1, is2, A, oh, mcol,
                      lb_w, la1_w, la2_w, lb1_w, lb2_w,
                      l1_w, l1_b.reshape(1, H), l2_w, l2_b.reshape(1, H),
                      l3_w, pos_emb_w)

    return _run_scores(item_emb_w, hf)


# speedup vs baseline: 65.5913x; 1.1335x over previous
import jax
import jax.numpy as jnp
import numpy as np
from jax.experimental import pallas as pl
from jax.experimental.pallas import tpu as pltpu

N_ITEMS = 100000
N_INTENTS = 512
H = 64
B = 64
L = 50
NU = 50
ALPHA = 0.5

RB1 = 1000
NBLK1 = N_ITEMS // RB1
RB3 = 2048
NBLK3 = (N_ITEMS + RB3 - 1) // RB3

_F32 = jnp.float32
_NEG = -3e38


def _dot(a, b, ca, cb):
    return jax.lax.dot_general(
        a, b, (((ca,), (cb,)), ((), ())), preferred_element_type=_F32)


def _leaky(x):
    return jnp.where(x >= 0, x, 0.2 * x)


def _graph_kernel(item_ref, iw_ref, k_ref, lq_ref, la_ref,
                  topi_ref, inew_ref, m_ref, s_ref, v_ref):
    i = pl.program_id(0)

    @pl.when(i == 0)
    def _init():
        m_ref[...] = jnp.full((1, N_INTENTS), _NEG, _F32)
        s_ref[...] = jnp.zeros((1, N_INTENTS), _F32)
        v_ref[...] = jnp.zeros((H, N_INTENTS), _F32)

    blk = item_ref[...]
    w = blk / jnp.sqrt(jnp.sum(blk * blk, axis=1, keepdims=True))
    iw = iw_ref[...]
    kmat = k_ref[...]

    q = _dot(w, lq_ref[...], 1, 1)
    logits = _dot(q, kmat, 1, 1)

    ic = jax.lax.broadcasted_iota(
        jnp.int32, (N_INTENTS, 1), 0).astype(_F32)
    l = logits
    sel_any = jnp.zeros((RB1, N_INTENTS), jnp.bool_)
    idxs = []
    for _ in range(3):
        mk = jnp.max(l, axis=1, keepdims=True)
        sel = l == mk
        sel_any = jnp.logical_or(sel_any, sel)
        self_f = jnp.where(sel, 1.0, 0.0)
        l = jnp.where(sel, _NEG, l)
        idxs.append(_dot(self_f, ic, 1, 0))
    topi_ref[...] = jnp.concatenate(idxs, axis=1).astype(jnp.int32)

    p = _dot(w * la_ref[...], iw, 1, 1)
    em = jnp.where(sel_any, _leaky(p), _NEG)
    bm = jnp.max(em, axis=0, keepdims=True)
    m_old = m_ref[...]
    m_new = jnp.maximum(m_old, bm)
    eexp = jnp.exp(em - m_new)
    scale = jnp.exp(m_old - m_new)
    s_ref[...] = s_ref[...] * scale + jnp.sum(eexp, axis=0, keepdims=True)
    v_ref[...] = v_ref[...] * scale + _dot(w, eexp, 0, 0)
    m_ref[...] = m_new

    @pl.when(i == NBLK1 - 1)
    def _fin():
        s = s_ref[...]
        inew_ref[...] = jnp.where(s > 0, v_ref[...] / s, 0.0)


def _run_graph(item_emb_w, iw, kmat, lq_w, la_w):
    topi, inew = pl.pallas_call(
        _graph_kernel,
        grid=(NBLK1,),
        in_specs=[
            pl.BlockSpec((RB1, H), lambda i: (i, 0)),
            pl.BlockSpec((N_INTENTS, H), lambda i: (0, 0)),
            pl.BlockSpec((N_INTENTS, H), lambda i: (0, 0)),
            pl.BlockSpec((H, H), lambda i: (0, 0)),
            pl.BlockSpec((1, H), lambda i: (0, 0)),
        ],
        out_specs=[
            pl.BlockSpec((RB1, 3), lambda i: (i, 0)),
            pl.BlockSpec((H, N_INTENTS), lambda i: (0, 0)),
        ],
        out_shape=[
            jax.ShapeDtypeStruct((N_ITEMS, 3), jnp.int32),
            jax.ShapeDtypeStruct((H, N_INTENTS), _F32),
        ],
        scratch_shapes=[
            pltpu.VMEM((1, N_INTENTS), _F32),
            pltpu.VMEM((1, N_INTENTS), _F32),
            pltpu.VMEM((H, N_INTENTS), _F32),
        ],
    )(item_emb_w, iw, kmat, lq_w, la_w)
    return topi, inew


def _session_kernel(hi_ref, win_ref, is0_ref, is1_ref, is2_ref, a_ref, oh_ref,
                    mcol_ref, lb_ref, la1_ref, la2_ref, lb1_ref, lb2_ref,
                    l1w_ref, l1b_ref, l2w_ref, l2b_ref, l3w_ref, pos_ref,
                    hf_ref):
    hi_raw = hi_ref[0]
    hi = hi_raw / jnp.sqrt(jnp.sum(hi_raw * hi_raw, axis=1, keepdims=True))
    win_raw = win_ref[0]
    win = win_raw / jnp.sqrt(jnp.sum(win_raw * win_raw, axis=1, keepdims=True))
    mcol = mcol_ref[0]
    msum = jnp.sum(mcol)
    hs = jnp.sum(win * mcol, axis=0, keepdims=True) / msum

    lb = lb_ref[...]
    is0, is1, is2 = is0_ref[0], is1_ref[0], is2_ref[0]
    e0 = _leaky(jnp.sum(hi * is0 * lb, axis=1, keepdims=True))
    e1 = _leaky(jnp.sum(hi * is1 * lb, axis=1, keepdims=True))
    e2 = _leaky(jnp.sum(hi * is2 * lb, axis=1, keepdims=True))
    em = jnp.maximum(e0, jnp.maximum(e1, e2))
    x0, x1, x2 = jnp.exp(e0 - em), jnp.exp(e1 - em), jnp.exp(e2 - em)
    xs = x0 + x1 + x2
    item_nei = (x0 * is0 + x1 * is1 + x2 * is2) / xs
    h_intent = ALPHA * hi + (1.0 - ALPHA) * item_nei

    a = a_ref[0]
    c1 = _dot(hs * lb1_ref[...], hi, 1, 1)
    c2 = _dot(hs * lb2_ref[...], hi, 1, 1)
    att1 = _leaky(_dot(hi * la1_ref[...], hi, 1, 1) + c1)
    att2 = _leaky(_dot(hi * la2_ref[...], hi, 1, 1) + c2)
    al = jnp.where(a == 1.0, att1, jnp.where(a == 2.0, att2, -9e15))
    almax = jnp.max(al, axis=1, keepdims=True)
    ale = jnp.exp(al - almax)
    al = ale / jnp.sum(ale, axis=1, keepdims=True)
    h_local = _dot(al, hi, 1, 0)

    output = h_local + h_intent + h_local * h_intent
    sessions = _dot(oh_ref[0], output, 1, 0)

    spe = pos_ref[0:L, :]
    hi2 = sessions + spe
    hs2 = jnp.sum(hi2 * mcol, axis=0, keepdims=True) / msum
    q1 = _dot(hi2, l1w_ref[...], 1, 1) + l1b_ref[...]
    q2 = _dot(hs2, l2w_ref[...], 1, 1) + l2b_ref[...]
    alpha_r = _dot(jax.nn.sigmoid(q1 + q2), l3w_ref[...], 1, 1)
    hf = jnp.sum(alpha_r * sessions * mcol, axis=0, keepdims=True)
    pos_n = pos_ref[L:L + 1, :] * (
        jnp.sum(spe * mcol, axis=0, keepdims=True) / msum)
    hf_ref[0] = hf + pos_n


def _run_session(hi_raw, win_raw, is0, is1, is2, a, oh, mcol,
                 lb_w, la1_w, la2_w, lb1_w, lb2_w,
                 l1_w, l1_b, l2_w, l2_b, l3_w, pos_emb_w):
    full = lambda s: pl.BlockSpec(s, lambda b: tuple(0 for _ in s))
    per_b3 = lambda s1, s2: pl.BlockSpec((1, s1, s2), lambda b: (b, 0, 0))
    hf = pl.pallas_call(
        _session_kernel,
        grid=(B,),
        in_specs=[
            per_b3(NU, H), per_b3(L, H),
            per_b3(NU, H), per_b3(NU, H), per_b3(NU, H),
            per_b3(NU, NU), per_b3(L, NU), per_b3(L, 1),
            full((1, H)), full((1, H)), full((1, H)), full((1, H)),
            full((1, H)),
            full((H, H)), full((1, H)), full((H, H)), full((1, H)),
            full((1, H)), full((L + 1, H)),
        ],
        out_specs=pl.BlockSpec((1, 1, H), lambda b: (b, 0, 0)),
        out_shape=jax.ShapeDtypeStruct((B, 1, H), _F32),
    )(hi_raw, win_raw, is0, is1, is2, a, oh, mcol,
      lb_w, la1_w, la2_w, lb1_w, lb2_w,
      l1_w, l1_b, l2_w, l2_b, l3_w, pos_emb_w)
    return hf.reshape(B, H)


def _scores_kernel(item_ref, hf_ref, out_ref):
    blk = item_ref[...]
    w = blk / jnp.sqrt(jnp.sum(blk * blk, axis=1, keepdims=True))
    out_ref[...] = _dot(hf_ref[...], w, 1, 1)


def _run_scores(item_emb_w, hf):
    full = pl.pallas_call(
        _scores_kernel,
        grid=(NBLK3,),
        in_specs=[
            pl.BlockSpec((RB3, H), lambda j: (j, 0)),
            pl.BlockSpec((B, H), lambda j: (0, 0)),
        ],
        out_specs=pl.BlockSpec((B, RB3), lambda j: (0, j)),
        out_shape=jax.ShapeDtypeStruct((B, N_ITEMS), _F32),
    )(item_emb_w, hf)
    return full[:, 1:]


def kernel(items, A, inputs, masks, alias_inputs, item_emb_w, intent_emb_w,
           pos_emb_w, lq_w, lk_w, la_w, lb_w, la1_w, la2_w, lb1_w, lb2_w,
           l1_w, l1_b, l2_w, l2_b, l3_w):
    iw = intent_emb_w / jnp.linalg.norm(intent_emb_w, axis=1, keepdims=True)
    kmat = (iw @ lk_w.T) * np.float32(1.0 / np.sqrt(H))

    topi, inew = _run_graph(item_emb_w, iw, kmat, lq_w, la_w)
    intent_new = inew.T

    items_f = items.reshape(-1)
    hi_raw = jnp.take(item_emb_w, items_f, axis=0).reshape(B, NU, H)
    win_raw = jnp.take(item_emb_w, inputs.reshape(-1), axis=0).reshape(B, L, H)
    ti = jnp.take(topi, items_f, axis=0)
    is0 = jnp.take(intent_new, ti[:, 0], axis=0).reshape(B, NU, H)
    is1 = jnp.take(intent_new, ti[:, 1], axis=0).reshape(B, NU, H)
    is2 = jnp.take(intent_new, ti[:, 2], axis=0).reshape(B, NU, H)
    oh = jax.nn.one_hot(alias_inputs, NU, dtype=_F32)
    mcol = masks.reshape(B, L, 1)

    hf = _run_session(hi_raw, win_raw, is0, is1, is2, A, oh, mcol,
                      lb_w, la1_w, la2_w, lb1_w, lb2_w,
                      l1_w, l1_b.reshape(1, H), l2_w, l2_b.reshape(1, H),
                      l3_w, pos_emb_w)

    return _run_scores(item_emb_w, hf)


# speedup vs baseline: 75.4797x; 1.1508x over previous
import jax
import jax.numpy as jnp
import numpy as np
from jax.experimental import pallas as pl
from jax.experimental.pallas import tpu as pltpu

N_ITEMS = 100000
N_INTENTS = 512
H = 64
B = 64
L = 50
NU = 50
ALPHA = 0.5

RB1 = 2000
NBLK1 = N_ITEMS // RB1
RB3 = 4096
NBLK3 = (N_ITEMS + RB3 - 1) // RB3

_F32 = jnp.float32
_NEG = -3e38


def _dot(a, b, ca, cb):
    return jax.lax.dot_general(
        a, b, (((ca,), (cb,)), ((), ())), preferred_element_type=_F32)


def _leaky(x):
    return jnp.where(x >= 0, x, 0.2 * x)


def _graph_kernel(item_ref, iw_ref, k_ref, lq_ref, la_ref,
                  topi_ref, inew_ref, m_ref, s_ref, v_ref):
    i = pl.program_id(0)

    @pl.when(i == 0)
    def _init():
        m_ref[...] = jnp.full((1, N_INTENTS), _NEG, _F32)
        s_ref[...] = jnp.zeros((1, N_INTENTS), _F32)
        v_ref[...] = jnp.zeros((H, N_INTENTS), _F32)

    blk = item_ref[...]
    w = blk / jnp.sqrt(jnp.sum(blk * blk, axis=1, keepdims=True))
    iw = iw_ref[...]
    kmat = k_ref[...]

    q = _dot(w, lq_ref[...], 1, 1)
    logits = _dot(q, kmat, 1, 1)

    ic = jax.lax.broadcasted_iota(
        jnp.int32, (N_INTENTS, 1), 0).astype(_F32)
    l = logits
    m1 = jnp.max(l, axis=1, keepdims=True)
    s1 = l == m1
    lm1 = jnp.where(s1, _NEG, l)
    m2 = jnp.max(lm1, axis=1, keepdims=True)
    s2 = lm1 == m2
    lm2 = jnp.where(s2, _NEG, lm1)
    m3 = jnp.max(lm2, axis=1, keepdims=True)
    s3 = lm2 == m3
    sel_any = l >= m3
    i1 = _dot(jnp.where(s1, 1.0, 0.0), ic, 1, 0)
    i2 = _dot(jnp.where(s2, 1.0, 0.0), ic, 1, 0)
    i3 = _dot(jnp.where(s3, 1.0, 0.0), ic, 1, 0)
    topi_ref[...] = jnp.concatenate([i1, i2, i3], axis=1).astype(jnp.int32)

    p = _dot(w * la_ref[...], iw, 1, 1)
    em = jnp.where(sel_any, _leaky(p), _NEG)
    bm = jnp.max(em, axis=0, keepdims=True)
    m_old = m_ref[...]
    m_new = jnp.maximum(m_old, bm)
    eexp = jnp.exp(em - m_new)
    scale = jnp.exp(m_old - m_new)
    s_ref[...] = s_ref[...] * scale + jnp.sum(eexp, axis=0, keepdims=True)
    v_ref[...] = v_ref[...] * scale + _dot(w, eexp, 0, 0)
    m_ref[...] = m_new

    @pl.when(i == NBLK1 - 1)
    def _fin():
        s = s_ref[...]
        inew_ref[...] = jnp.where(s > 0, v_ref[...] / s, 0.0)


def _run_graph(item_emb_w, iw, kmat, lq_w, la_w):
    topi, inew = pl.pallas_call(
        _graph_kernel,
        grid=(NBLK1,),
        in_specs=[
            pl.BlockSpec((RB1, H), lambda i: (i, 0)),
            pl.BlockSpec((N_INTENTS, H), lambda i: (0, 0)),
            pl.BlockSpec((N_INTENTS, H), lambda i: (0, 0)),
            pl.BlockSpec((H, H), lambda i: (0, 0)),
            pl.BlockSpec((1, H), lambda i: (0, 0)),
        ],
        out_specs=[
            pl.BlockSpec((RB1, 3), lambda i: (i, 0)),
            pl.BlockSpec((H, N_INTENTS), lambda i: (0, 0)),
        ],
        out_shape=[
            jax.ShapeDtypeStruct((N_ITEMS, 3), jnp.int32),
            jax.ShapeDtypeStruct((H, N_INTENTS), _F32),
        ],
        scratch_shapes=[
            pltpu.VMEM((1, N_INTENTS), _F32),
            pltpu.VMEM((1, N_INTENTS), _F32),
            pltpu.VMEM((H, N_INTENTS), _F32),
        ],
    )(item_emb_w, iw, kmat, lq_w, la_w)
    return topi, inew


def _session_kernel(hi_ref, win_ref, is0_ref, is1_ref, is2_ref, a_ref, oh_ref,
                    mcol_ref, lb_ref, la1_ref, la2_ref, lb1_ref, lb2_ref,
                    l1w_ref, l1b_ref, l2w_ref, l2b_ref, l3w_ref, pos_ref,
                    hf_ref):
    hi_raw = hi_ref[0]
    hi = hi_raw / jnp.sqrt(jnp.sum(hi_raw * hi_raw, axis=1, keepdims=True))
    win_raw = win_ref[0]
    win = win_raw / jnp.sqrt(jnp.sum(win_raw * win_raw, axis=1, keepdims=True))
    mcol = mcol_ref[0]
    msum = jnp.sum(mcol)
    hs = jnp.sum(win * mcol, axis=0, keepdims=True) / msum

    lb = lb_ref[...]
    is0, is1, is2 = is0_ref[0], is1_ref[0], is2_ref[0]
    e0 = _leaky(jnp.sum(hi * is0 * lb, axis=1, keepdims=True))
    e1 = _leaky(jnp.sum(hi * is1 * lb, axis=1, keepdims=True))
    e2 = _leaky(jnp.sum(hi * is2 * lb, axis=1, keepdims=True))
    em = jnp.maximum(e0, jnp.maximum(e1, e2))
    x0, x1, x2 = jnp.exp(e0 - em), jnp.exp(e1 - em), jnp.exp(e2 - em)
    xs = x0 + x1 + x2
    item_nei = (x0 * is0 + x1 * is1 + x2 * is2) / xs
    h_intent = ALPHA * hi + (1.0 - ALPHA) * item_nei

    a = a_ref[0]
    c1 = _dot(hs * lb1_ref[...], hi, 1, 1)
    c2 = _dot(hs * lb2_ref[...], hi, 1, 1)
    att1 = _leaky(_dot(hi * la1_ref[...], hi, 1, 1) + c1)
    att2 = _leaky(_dot(hi * la2_ref[...], hi, 1, 1) + c2)
    al = jnp.where(a == 1.0, att1, jnp.where(a == 2.0, att2, -9e15))
    almax = jnp.max(al, axis=1, keepdims=True)
    ale = jnp.exp(al - almax)
    al = ale / jnp.sum(ale, axis=1, keepdims=True)
    h_local = _dot(al, hi, 1, 0)

    output = h_local + h_intent + h_local * h_intent
    sessions = _dot(oh_ref[0], output, 1, 0)

    spe = pos_ref[0:L, :]
    hi2 = sessions + spe
    hs2 = jnp.sum(hi2 * mcol, axis=0, keepdims=True) / msum
    q1 = _dot(hi2, l1w_ref[...], 1, 1) + l1b_ref[...]
    q2 = _dot(hs2, l2w_ref[...], 1, 1) + l2b_ref[...]
    alpha_r = _dot(jax.nn.sigmoid(q1 + q2), l3w_ref[...], 1, 1)
    hf = jnp.sum(alpha_r * sessions * mcol, axis=0, keepdims=True)
    pos_n = pos_ref[L:L + 1, :] * (
        jnp.sum(spe * mcol, axis=0, keepdims=True) / msum)
    hf_ref[0] = hf + pos_n


def _run_session(hi_raw, win_raw, is0, is1, is2, a, oh, mcol,
                 lb_w, la1_w, la2_w, lb1_w, lb2_w,
                 l1_w, l1_b, l2_w, l2_b, l3_w, pos_emb_w):
    full = lambda s: pl.BlockSpec(s, lambda b: tuple(0 for _ in s))
    per_b3 = lambda s1, s2: pl.BlockSpec((1, s1, s2), lambda b: (b, 0, 0))
    hf = pl.pallas_call(
        _session_kernel,
        grid=(B,),
        in_specs=[
            per_b3(NU, H), per_b3(L, H),
            per_b3(NU, H), per_b3(NU, H), per_b3(NU, H),
            per_b3(NU, NU), per_b3(L, NU), per_b3(L, 1),
            full((1, H)), full((1, H)), full((1, H)), full((1, H)),
            full((1, H)),
            full((H, H)), full((1, H)), full((H, H)), full((1, H)),
            full((1, H)), full((L + 1, H)),
        ],
        out_specs=pl.BlockSpec((1, 1, H), lambda b: (b, 0, 0)),
        out_shape=jax.ShapeDtypeStruct((B, 1, H), _F32),
    )(hi_raw, win_raw, is0, is1, is2, a, oh, mcol,
      lb_w, la1_w, la2_w, lb1_w, lb2_w,
      l1_w, l1_b, l2_w, l2_b, l3_w, pos_emb_w)
    return hf.reshape(B, H)


def _scores_kernel(item_ref, hf_ref, out_ref):
    blk = item_ref[...]
    w = blk / jnp.sqrt(jnp.sum(blk * blk, axis=1, keepdims=True))
    out_ref[...] = _dot(hf_ref[...], w, 1, 1)


def _run_scores(item_emb_w, hf):
    full = pl.pallas_call(
        _scores_kernel,
        grid=(NBLK3,),
        in_specs=[
            pl.BlockSpec((RB3, H), lambda j: (j, 0)),
            pl.BlockSpec((B, H), lambda j: (0, 0)),
        ],
        out_specs=pl.BlockSpec((B, RB3), lambda j: (0, j)),
        out_shape=jax.ShapeDtypeStruct((B, N_ITEMS), _F32),
    )(item_emb_w, hf)
    return full[:, 1:]


def kernel(items, A, inputs, masks, alias_inputs, item_emb_w, intent_emb_w,
           pos_emb_w, lq_w, lk_w, la_w, lb_w, la1_w, la2_w, lb1_w, lb2_w,
           l1_w, l1_b, l2_w, l2_b, l3_w):
    iw = intent_emb_w / jnp.linalg.norm(intent_emb_w, axis=1, keepdims=True)
    kmat = (iw @ lk_w.T) * np.float32(1.0 / np.sqrt(H))

    topi, inew = _run_graph(item_emb_w, iw, kmat, lq_w, la_w)
    intent_new = inew.T

    items_f = items.reshape(-1)
    rows = jnp.concatenate([items_f, inputs.reshape(-1)])
    g = jnp.take(item_emb_w, rows, axis=0)
    hi_raw = g[:B * NU].reshape(B, NU, H)
    win_raw = g[B * NU:].reshape(B, L, H)
    ti = jnp.take(topi, items_f, axis=0)
    isx = jnp.take(intent_new, ti.reshape(-1), axis=0)
    isx = isx.reshape(B * NU, 3, H)
    is0 = isx[:, 0].reshape(B, NU, H)
    is1 = isx[:, 1].reshape(B, NU, H)
    is2 = isx[:, 2].reshape(B, NU, H)
    oh = jax.nn.one_hot(alias_inputs, NU, dtype=_F32)
    mcol = masks.reshape(B, L, 1)

    hf = _run_session(hi_raw, win_raw, is0, is1, is2, A, oh, mcol,
                      lb_w, la1_w, la2_w, lb1_w, lb2_w,
                      l1_w, l1_b.reshape(1, H), l2_w, l2_b.reshape(1, H),
                      l3_w, pos_emb_w)

    return _run_scores(item_emb_w, hf)
